# baseline (speedup 1.0000x reference)
import functools

import jax
import jax.numpy as jnp
from jax import lax
from jax.experimental import pallas as pl
from jax.experimental.pallas import tpu as pltpu
from jax.experimental.pallas import tpu_sc as plsc

B = 16384
D = 64
U = 100000
NC, NS = 2, 16
NW = NC * NS
BPW = B // NW
GC = 128
NGC = BPW // GC
RS = 1000

_sc_mesh = plsc.VectorSubcoreMesh(core_axis_name="c", subcore_axis_name="s")


def _spread_body(ut_ref, it_ref, uo_ref, io_ref):
    uo_ref[:, 0:D] = ut_ref[...]
    uo_ref[:, D:2 * D] = jnp.zeros((RS, D), jnp.float32)
    io_ref[:, 0:D] = it_ref[...]
    io_ref[:, D:2 * D] = jnp.zeros((RS, D), jnp.float32)


def _spread(ut, it):
    return pl.pallas_call(
        _spread_body,
        grid=(U // RS,),
        in_specs=[
            pl.BlockSpec((RS, D), lambda j: (j, 0)),
            pl.BlockSpec((RS, D), lambda j: (j, 0)),
        ],
        out_specs=[
            pl.BlockSpec((RS, 2 * D), lambda j: (j, 0)),
            pl.BlockSpec((RS, 2 * D), lambda j: (j, 0)),
        ],
        out_shape=[
            jax.ShapeDtypeStruct((U, 2 * D), jnp.float32),
            jax.ShapeDtypeStruct((U, 2 * D), jnp.float32),
        ],
    )(ut, it)


@functools.partial(
    pl.kernel,
    mesh=_sc_mesh,
    out_type=(
        jax.ShapeDtypeStruct((B, 2 * D), jnp.float32),
        jax.ShapeDtypeStruct((B, 2 * D), jnp.float32),
    ),
    scratch_types=[
        pltpu.VMEM((BPW,), jnp.int32),
        pltpu.VMEM((BPW,), jnp.int32),
        pltpu.VMEM((GC, 2 * D), jnp.float32),
        pltpu.VMEM((GC, 2 * D), jnp.float32),
        pltpu.VMEM((GC, 2 * D), jnp.float32),
        pltpu.VMEM((GC, 2 * D), jnp.float32),
        pltpu.VMEM((GC, 2 * D), jnp.float32),
        pltpu.VMEM((GC, 2 * D), jnp.float32),
        pltpu.SemaphoreType.DMA,
        pltpu.SemaphoreType.DMA,
        pltpu.SemaphoreType.DMA,
    ],
)
def _sc_gather(uid_hbm, iid_hbm, ucat, icat, xu_hbm, xi_hbm,
               uidx_v, iidx_v, ubuf0, ubuf1, ubuf2, ibuf0, ibuf1, ibuf2,
               usem, isem, wsem):
    ubuf = (ubuf0, ubuf1, ubuf2)
    ibuf = (ibuf0, ibuf1, ibuf2)

    wid = lax.axis_index("s") * NC + lax.axis_index("c")
    base = wid * BPW
    pltpu.sync_copy(uid_hbm.at[pl.ds(base, BPW)], uidx_v)
    pltpu.sync_copy(iid_hbm.at[pl.ds(base, BPW)], iidx_v)

    def fire(k):
        b = k % 3
        pltpu.async_copy(ucat.at[uidx_v.at[pl.ds(k * GC, GC)]], ubuf[b], usem)
        pltpu.async_copy(icat.at[iidx_v.at[pl.ds(k * GC, GC)]], ibuf[b], isem)

    def wait_gather(k):
        b = k % 3
        pltpu.make_async_copy(ucat.at[uidx_v.at[pl.ds(k * GC, GC)]],
                              ubuf[b], usem).wait()
        pltpu.make_async_copy(icat.at[iidx_v.at[pl.ds(k * GC, GC)]],
                              ibuf[b], isem).wait()

    def rows(k):
        return pl.ds(base + k * GC, GC)

    def writeback(k):
        b = k % 3
        pltpu.async_copy(ubuf[b], xu_hbm.at[rows(k)], wsem)
        pltpu.async_copy(ibuf[b], xi_hbm.at[rows(k)], wsem)

    def writeback_wait(k):
        b = k % 3
        pltpu.make_async_copy(ubuf[b], xu_hbm.at[rows(k)], wsem).wait()
        pltpu.make_async_copy(ibuf[b], xi_hbm.at[rows(k)], wsem).wait()

    fire(0)
    if NGC > 1:
        fire(1)
    for k in range(NGC):
        wait_gather(k)
        if k >= 1:
            writeback_wait(k - 1)
        writeback(k)
        if k + 2 < NGC:
            fire(k + 2)
    writeback_wait(NGC - 1)


BM = 2048
NB = B // BM


def _mlp_body(xu_ref, xi_ref, p1_ref, p2_ref, b1_ref, w2t_ref, b2_ref,
              w3_ref, b3_ref, o_ref):
    h = jnp.dot(xu_ref[...], p1_ref[...], preferred_element_type=jnp.float32)
    h = h + jnp.dot(xi_ref[...], p2_ref[...], preferred_element_type=jnp.float32)
    h = jnp.maximum(h + b1_ref[...], 0.0)
    h2 = jnp.dot(h, w2t_ref[...], preferred_element_type=jnp.float32)
    h2 = jnp.maximum(h2 + b2_ref[...], 0.0)
    o_ref[...] = jnp.sum(h2 * w3_ref[...], axis=1) + b3_ref[0, 0]


def _mlp(xu, xi, p1, p2, b1, w2_t, b2, w3, b3):
    return pl.pallas_call(
        _mlp_body,
        grid=(NB,),
        in_specs=[
            pl.BlockSpec((BM, 2 * D), lambda j: (j, 0)),
            pl.BlockSpec((BM, 2 * D), lambda j: (j, 0)),
            pl.BlockSpec((2 * D, 128), lambda j: (0, 0)),
            pl.BlockSpec((2 * D, 128), lambda j: (0, 0)),
            pl.BlockSpec((1, 128), lambda j: (0, 0)),
            pl.BlockSpec((128, D), lambda j: (0, 0)),
            pl.BlockSpec((1, D), lambda j: (0, 0)),
            pl.BlockSpec((1, D), lambda j: (0, 0)),
            pl.BlockSpec((1, 1), lambda j: (0, 0)),
        ],
        out_specs=pl.BlockSpec((BM,), lambda j: (j,)),
        out_shape=jax.ShapeDtypeStruct((B,), jnp.float32),
    )(xu, xi, p1, p2, b1, w2_t, b2, w3, b3)


def kernel(user_ids, item_ids, user_table, item_table, W1, b1, W2, b2, W3, b3):
    ucat, icat = _spread(user_table, item_table)
    xu, xi = _sc_gather(user_ids.astype(jnp.int32), item_ids.astype(jnp.int32),
                        ucat, icat)
    w1t = W1.T
    p1 = w1t
    p2 = jnp.roll(w1t, -D, axis=0)
    out = _mlp(xu, xi, p1, p2, b1.reshape(1, 128), W2.T, b2.reshape(1, D),
               W3.reshape(1, D), b3.reshape(1, 1))
    return out


# speedup vs baseline: 1.5216x; 1.5216x over previous
import functools

import jax
import jax.numpy as jnp
from jax import lax
from jax.experimental import pallas as pl
from jax.experimental.pallas import tpu as pltpu
from jax.experimental.pallas import tpu_sc as plsc

B = 16384
D = 64
NC, NS = 2, 16
NW = NC * NS
BPW = B // NW

_sc_mesh = plsc.VectorSubcoreMesh(core_axis_name="c", subcore_axis_name="s")


@functools.partial(
    pl.kernel,
    mesh=_sc_mesh,
    out_type=jax.ShapeDtypeStruct((B, 2 * D), jnp.float32),
    scratch_types=[
        pltpu.VMEM((BPW,), jnp.int32),
        pltpu.VMEM((BPW,), jnp.int32),
        pltpu.VMEM((BPW, D), jnp.float32),
        pltpu.VMEM((BPW, D), jnp.float32),
        pltpu.SemaphoreType.DMA,
        pltpu.SemaphoreType.DMA,
    ],
    compiler_params=pltpu.CompilerParams(use_tc_tiling_on_sc=False),
)
def _sc_gather(uid_hbm, iid_hbm, utab_hbm, itab_hbm, out_hbm,
               uidx_v, iidx_v, urows_v, irows_v, usem, isem):
    wid = lax.axis_index("s") * NC + lax.axis_index("c")
    base = wid * BPW
    pltpu.sync_copy(uid_hbm.at[pl.ds(base, BPW)], uidx_v)
    ucp = pltpu.async_copy(utab_hbm.at[uidx_v], urows_v, usem)
    pltpu.sync_copy(iid_hbm.at[pl.ds(base, BPW)], iidx_v)
    icp = pltpu.async_copy(itab_hbm.at[iidx_v], irows_v, isem)
    ucp.wait()
    pltpu.sync_copy(urows_v, out_hbm.at[pl.ds(base, BPW), pl.ds(0, D)])
    icp.wait()
    pltpu.sync_copy(irows_v, out_hbm.at[pl.ds(base, BPW), pl.ds(D, D)])


BM = 4096
NB = B // BM


def _mlp_body(x_ref, w1t_ref, b1_ref, w2t_ref, b2_ref, w3_ref, b3_ref, o_ref):
    x16 = x_ref[...].astype(jnp.bfloat16)
    h = jnp.dot(x16, w1t_ref[...], preferred_element_type=jnp.float32)
    h = jnp.maximum(h + b1_ref[...], 0.0)
    h2 = jnp.dot(h.astype(jnp.bfloat16), w2t_ref[...],
                 preferred_element_type=jnp.float32)
    h2 = jnp.maximum(h2 + b2_ref[...], 0.0)
    o_ref[...] = jnp.sum(h2 * w3_ref[...], axis=1) + b3_ref[0, 0]


def _mlp(x, w1_t, b1, w2_t, b2, w3, b3):
    return pl.pallas_call(
        _mlp_body,
        grid=(NB,),
        in_specs=[
            pl.BlockSpec((BM, 2 * D), lambda j: (j, 0)),
            pl.BlockSpec((2 * D, 128), lambda j: (0, 0)),
            pl.BlockSpec((1, 128), lambda j: (0, 0)),
            pl.BlockSpec((128, D), lambda j: (0, 0)),
            pl.BlockSpec((1, D), lambda j: (0, 0)),
            pl.BlockSpec((1, D), lambda j: (0, 0)),
            pl.BlockSpec((1, 1), lambda j: (0, 0)),
        ],
        out_specs=pl.BlockSpec((BM,), lambda j: (j,)),
        out_shape=jax.ShapeDtypeStruct((B,), jnp.float32),
    )(x, w1_t, b1, w2_t, b2, w3, b3)


def kernel(user_ids, item_ids, user_table, item_table, W1, b1, W2, b2, W3, b3):
    x = _sc_gather(user_ids.astype(jnp.int32), item_ids.astype(jnp.int32),
                   user_table, item_table)
    out = _mlp(x, W1.T.astype(jnp.bfloat16), b1.reshape(1, 128),
               W2.T.astype(jnp.bfloat16), b2.reshape(1, D),
               W3.reshape(1, D), b3.reshape(1, 1))
    return out


# speedup vs baseline: 1.5355x; 1.0091x over previous
import functools

import jax
import jax.numpy as jnp
from jax import lax
from jax.experimental import pallas as pl
from jax.experimental.pallas import tpu as pltpu
from jax.experimental.pallas import tpu_sc as plsc

B = 16384
D = 64
NC, NS = 2, 16
NW = NC * NS
BPW = B // NW

_sc_mesh = plsc.VectorSubcoreMesh(core_axis_name="c", subcore_axis_name="s")


@functools.partial(
    pl.kernel,
    mesh=_sc_mesh,
    out_type=jax.ShapeDtypeStruct((B, 2 * D), jnp.float32),
    scratch_types=[
        pltpu.VMEM((BPW,), jnp.int32),
        pltpu.VMEM((BPW,), jnp.int32),
        pltpu.VMEM((BPW, D), jnp.float32),
        pltpu.VMEM((BPW, D), jnp.float32),
        pltpu.SemaphoreType.DMA,
        pltpu.SemaphoreType.DMA,
    ],
    compiler_params=pltpu.CompilerParams(use_tc_tiling_on_sc=False),
)
def _sc_gather(uid_hbm, iid_hbm, utab_hbm, itab_hbm, out_hbm,
               uidx_v, iidx_v, urows_v, irows_v, usem, isem):
    wid = lax.axis_index("s") * NC + lax.axis_index("c")
    base = wid * BPW
    pltpu.sync_copy(uid_hbm.at[pl.ds(base, BPW)], uidx_v)
    ucp = pltpu.async_copy(utab_hbm.at[uidx_v], urows_v, usem)
    pltpu.sync_copy(iid_hbm.at[pl.ds(base, BPW)], iidx_v)
    icp = pltpu.async_copy(itab_hbm.at[iidx_v], irows_v, isem)
    ucp.wait()
    pltpu.sync_copy(urows_v, out_hbm.at[pl.ds(base, BPW), pl.ds(0, D)])
    icp.wait()
    pltpu.sync_copy(irows_v, out_hbm.at[pl.ds(base, BPW), pl.ds(D, D)])


BM = 8192
NB = B // BM


def _mlp_body(x_ref, w1t_ref, b1_ref, w2t_ref, b2_ref, w3_ref, b3_ref, o_ref):
    x16 = x_ref[...].astype(jnp.bfloat16)
    h = jnp.dot(x16, w1t_ref[...], preferred_element_type=jnp.float32)
    h = jnp.maximum(h + b1_ref[...], 0.0)
    h2 = jnp.dot(h.astype(jnp.bfloat16), w2t_ref[...],
                 preferred_element_type=jnp.float32)
    h2 = jnp.maximum(h2 + b2_ref[...], 0.0)
    o_ref[...] = jnp.sum(h2 * w3_ref[...], axis=1) + b3_ref[0, 0]


def _mlp(x, w1_t, b1, w2_t, b2, w3, b3):
    return pl.pallas_call(
        _mlp_body,
        grid=(NB,),
        in_specs=[
            pl.BlockSpec((BM, 2 * D), lambda j: (j, 0)),
            pl.BlockSpec((2 * D, 128), lambda j: (0, 0)),
            pl.BlockSpec((1, 128), lambda j: (0, 0)),
            pl.BlockSpec((128, D), lambda j: (0, 0)),
            pl.BlockSpec((1, D), lambda j: (0, 0)),
            pl.BlockSpec((1, D), lambda j: (0, 0)),
            pl.BlockSpec((1, 1), lambda j: (0, 0)),
        ],
        out_specs=pl.BlockSpec((BM,), lambda j: (j,)),
        out_shape=jax.ShapeDtypeStruct((B,), jnp.float32),
    )(x, w1_t, b1, w2_t, b2, w3, b3)


def kernel(user_ids, item_ids, user_table, item_table, W1, b1, W2, b2, W3, b3):
    x = _sc_gather(user_ids.astype(jnp.int32), item_ids.astype(jnp.int32),
                   user_table, item_table)
    out = _mlp(x, W1.T.astype(jnp.bfloat16), b1.reshape(1, 128),
               W2.T.astype(jnp.bfloat16), b2.reshape(1, D),
               W3.reshape(1, D), b3.reshape(1, 1))
    return out
